# baseline (speedup 1.0000x reference)
import jax
import jax.numpy as jnp
from jax import lax
from jax.experimental import pallas as pl
from jax.experimental.pallas import tpu as pltpu
from jax.experimental.pallas import tpu_sc as plsc

EMBED_DIM = 300
TAIL_OFF = 256
OUT_W = 384
NUM_CORES = 2
NUM_SUBCORES = 16
NUM_WORKERS = NUM_CORES * NUM_SUBCORES
CHUNK = 128


def _gather_body(ids_hbm, table_hbm, tail_hbm, out_hbm, idx_v, rows_v, sem):
    chunks_per_w = ids_hbm.shape[1]
    wid = lax.axis_index("s") * NUM_CORES + lax.axis_index("c")
    base = wid * chunks_per_w
    pltpu.sync_copy(ids_hbm.at[wid], idx_v)

    def step(j, carry):
        idx_row = idx_v.at[j]
        cp0 = pltpu.async_copy(
            table_hbm.at[idx_row, pl.ds(0, 128)],
            rows_v.at[:, pl.ds(0, 128)], sem)
        cp1 = pltpu.async_copy(
            table_hbm.at[idx_row, pl.ds(128, 128)],
            rows_v.at[:, pl.ds(128, 128)], sem)
        cp2 = pltpu.async_copy(
            tail_hbm.at[idx_row],
            rows_v.at[:, pl.ds(256, 128)], sem)
        cp0.wait()
        cp1.wait()
        cp2.wait()
        pltpu.sync_copy(rows_v, out_hbm.at[pl.ds((base + j) * CHUNK, CHUNK)])
        return carry

    lax.fori_loop(0, chunks_per_w, step, 0)


def kernel(token_ids, glove_table):
    B, S = token_ids.shape
    V = glove_table.shape[0]
    n = B * S
    assert n % (NUM_WORKERS * CHUNK) == 0
    n_chunks = n // CHUNK
    chunks_per_w = n_chunks // NUM_WORKERS
    ids = token_ids.astype(jnp.int32).reshape(NUM_WORKERS, chunks_per_w, CHUNK)
    tail = jnp.pad(
        lax.slice(glove_table, (0, TAIL_OFF), (V, EMBED_DIM)),
        ((0, 0), (0, 128 - (EMBED_DIM - TAIL_OFF))))

    run = pl.kernel(
        _gather_body,
        out_type=jax.ShapeDtypeStruct((n, OUT_W), jnp.float32),
        mesh=plsc.VectorSubcoreMesh(core_axis_name="c", subcore_axis_name="s"),
        scratch_types=[
            pltpu.VMEM((chunks_per_w, CHUNK), jnp.int32),
            pltpu.VMEM((CHUNK, OUT_W), jnp.float32),
            pltpu.SemaphoreType.DMA,
        ],
    )
    wide = run(ids, glove_table, tail)
    return lax.slice(wide, (0, 0), (n, EMBED_DIM)).reshape(B, S, EMBED_DIM)
